# baseline (speedup 1.0000x reference)
import functools
import jax
import jax.numpy as jnp
from jax import lax
from jax.experimental import pallas as pl
from jax.experimental.pallas import tpu as pltpu
from jax.experimental.pallas import tpu_sc as plsc

_H = 32
_NVALS = 12
_B, _N = 32, 256
_PLANE = _N * _N
_CHUNK = 2048
_NCHUNK = _PLANE // _CHUNK


def _sc_body(dist_hbm, table_hbm, out_hbm, table_v, idx_v, out_v):
    nc = 2
    wid = lax.axis_index("s") * nc + lax.axis_index("c")
    pltpu.sync_copy(table_hbm, table_v)

    def chunk_body(c, _):
        src = wid * _PLANE + c * _CHUNK
        pltpu.sync_copy(dist_hbm.at[pl.ds(src, _CHUNK)], idx_v)

        def vec_body(i, _):
            d = idx_v[pl.ds(i * 16, 16)]
            sc = jnp.minimum(jnp.maximum(d + 1, 0), _NVALS - 1) * _H
            for h in range(_H):
                out_v[h, pl.ds(i * 16, 16)] = plsc.load_gather(table_v, [sc + h])
            return 0

        lax.fori_loop(0, _CHUNK // 16, vec_body, 0, unroll=4)

        for h in range(_H):
            dst = (wid * _H + h) * _PLANE + c * _CHUNK
            pltpu.sync_copy(out_v.at[h], out_hbm.at[pl.ds(dst, _CHUNK)])
        return 0

    lax.fori_loop(0, _NCHUNK, chunk_body, 0)


def kernel(dist_matrix, bias_embedding):
    B, N, _ = dist_matrix.shape
    dist_flat = dist_matrix.reshape(B * N * N)
    table_flat = bias_embedding.reshape(_NVALS * _H)

    mesh = plsc.VectorSubcoreMesh(core_axis_name="c", subcore_axis_name="s")
    k = functools.partial(
        pl.kernel,
        mesh=mesh,
        out_type=jax.ShapeDtypeStruct((B * _H * N * N,), jnp.float32),
        scratch_types=[
            pltpu.VMEM((_NVALS * _H,), jnp.float32),
            pltpu.VMEM((_CHUNK,), jnp.int32),
            pltpu.VMEM((_H, _CHUNK), jnp.float32),
        ],
        compiler_params=pltpu.CompilerParams(needs_layout_passes=False),
    )(_sc_body)
    out = k(dist_flat, table_flat)
    return out.reshape(B * _H, N, N)


# speedup vs baseline: 1.0057x; 1.0057x over previous
import functools
import jax
import jax.numpy as jnp
from jax import lax
from jax.experimental import pallas as pl
from jax.experimental.pallas import tpu as pltpu
from jax.experimental.pallas import tpu_sc as plsc

_H = 32
_NVALS = 12
_N = 256
_PLANE = _N * _N
_CHUNK = 2048
_NCHUNK = _PLANE // _CHUNK
_HG = 8
_NG = _H // _HG


def _sc_body(dist_hbm, table_hbm, out_hbm, table_v, idx0_v, idx1_v, scaled_v,
             out_v, sem_in0, sem_in1, sem_out):
    nc = 2
    wid = lax.axis_index("s") * nc + lax.axis_index("c")
    base_in = wid * _PLANE
    base_out = wid * _H * _PLANE
    pltpu.sync_copy(table_hbm, table_v)

    idx_bufs = (idx0_v, idx1_v)
    sem_ins = (sem_in0, sem_in1)

    pltpu.async_copy(dist_hbm.at[pl.ds(base_in, _CHUNK)], idx0_v, sem_in0)
    pltpu.async_copy(dist_hbm.at[pl.ds(base_in + _CHUNK, _CHUNK)], idx1_v, sem_in1)

    def pair_body(cc, _):
        for p in range(2):
            c = 2 * cc + p
            idx_v = idx_bufs[p]
            sem_in = sem_ins[p]

            pltpu.make_async_copy(
                dist_hbm.at[pl.ds(0, _CHUNK)], idx_v, sem_in).wait()

            def scale_body(i, _):
                d = idx_v[pl.ds(i * 16, 16)]
                sc = jnp.minimum(jnp.maximum(d + 1, 0), _NVALS - 1) * _H
                scaled_v[pl.ds(i * 16, 16)] = sc
                return 0

            lax.fori_loop(0, _CHUNK // 16, scale_body, 0, unroll=4)

            nxt = jnp.minimum(c + 2, _NCHUNK - 1)
            pltpu.async_copy(
                dist_hbm.at[pl.ds(base_in + nxt * _CHUNK, _CHUNK)], idx_v, sem_in)

            for g in range(_NG):
                @pl.when(c >= 1)
                def _drain():
                    for hh in range(_HG):
                        pltpu.make_async_copy(
                            out_v.at[g * _HG + hh],
                            out_hbm.at[pl.ds(0, _CHUNK)], sem_out.at[g]).wait()

                def gather_body(i, _):
                    s = scaled_v[pl.ds(i * 16, 16)]
                    for hh in range(_HG):
                        h = g * _HG + hh
                        out_v[h, pl.ds(i * 16, 16)] = plsc.load_gather(
                            table_v, [s + h])
                    return 0

                lax.fori_loop(0, _CHUNK // 16, gather_body, 0, unroll=2)

                for hh in range(_HG):
                    h = g * _HG + hh
                    dst = base_out + h * _PLANE + c * _CHUNK
                    pltpu.async_copy(
                        out_v.at[h], out_hbm.at[pl.ds(dst, _CHUNK)], sem_out.at[g])
        return 0

    lax.fori_loop(0, _NCHUNK // 2, pair_body, 0)

    for g in range(_NG):
        for hh in range(_HG):
            pltpu.make_async_copy(
                out_v.at[g * _HG + hh],
                out_hbm.at[pl.ds(0, _CHUNK)], sem_out.at[g]).wait()
    for p in range(2):
        pltpu.make_async_copy(
            dist_hbm.at[pl.ds(0, _CHUNK)], idx_bufs[p], sem_ins[p]).wait()


def kernel(dist_matrix, bias_embedding):
    B, N, _ = dist_matrix.shape
    dist_flat = dist_matrix.reshape(B * N * N)
    table_flat = bias_embedding.reshape(_NVALS * _H)

    mesh = plsc.VectorSubcoreMesh(core_axis_name="c", subcore_axis_name="s")
    k = functools.partial(
        pl.kernel,
        mesh=mesh,
        out_type=jax.ShapeDtypeStruct((B * _H * N * N,), jnp.float32),
        scratch_types=[
            pltpu.VMEM((_NVALS * _H,), jnp.float32),
            pltpu.VMEM((_CHUNK,), jnp.int32),
            pltpu.VMEM((_CHUNK,), jnp.int32),
            pltpu.VMEM((_CHUNK,), jnp.int32),
            pltpu.VMEM((_H, _CHUNK), jnp.float32),
            pltpu.SemaphoreType.DMA,
            pltpu.SemaphoreType.DMA,
            pltpu.SemaphoreType.DMA((_NG,)),
        ],
        compiler_params=pltpu.CompilerParams(needs_layout_passes=False),
    )(_sc_body)
    out = k(dist_flat, table_flat)
    return out.reshape(B * _H, N, N)


# speedup vs baseline: 3.7150x; 3.6940x over previous
import functools
import jax
import jax.numpy as jnp
from jax import lax
from jax.experimental import pallas as pl
from jax.experimental.pallas import tpu as pltpu
from jax.experimental.pallas import tpu_sc as plsc

_H = 32
_NVALS = 12
_N = 256
_PLANE = _N * _N
_CHUNK = 2048
_NCHUNK = _PLANE // _CHUNK
_HG = 8
_NG = _H // _HG


def _sc_body(dist_hbm, table_hbm, out_hbm, table_v, idx0_v, idx1_v,
             out_v, sem_in0, sem_in1, sem_out):
    nc = 2
    wid = lax.axis_index("s") * nc + lax.axis_index("c")
    base_in = wid * _PLANE
    base_out = wid * _H * _PLANE
    pltpu.sync_copy(table_hbm, table_v)

    idx_bufs = (idx0_v, idx1_v)
    sem_ins = (sem_in0, sem_in1)

    pltpu.async_copy(dist_hbm.at[pl.ds(base_in, _CHUNK)], idx0_v, sem_in0)
    pltpu.async_copy(dist_hbm.at[pl.ds(base_in + _CHUNK, _CHUNK)], idx1_v, sem_in1)

    def pair_body(cc, _):
        for p in range(2):
            c = 2 * cc + p
            idx_v = idx_bufs[p]
            sem_in = sem_ins[p]

            pltpu.make_async_copy(
                dist_hbm.at[pl.ds(0, _CHUNK)], idx_v, sem_in).wait()

            for g in range(_NG):
                @pl.when(c >= 1)
                def _drain():
                    for hh in range(_HG):
                        pltpu.make_async_copy(
                            out_v.at[g * _HG + hh],
                            out_hbm.at[pl.ds(0, _CHUNK)], sem_out.at[g]).wait()

                ths = [table_v[pl.ds((g * _HG + hh) * 16, 16)]
                       for hh in range(_HG)]

                def gather_body(i, _):
                    d = idx_v[pl.ds(i * 16, 16)]
                    k = jnp.minimum(jnp.maximum(d + 1, 0), _NVALS - 1)
                    for hh in range(_HG):
                        out_v[g * _HG + hh, pl.ds(i * 16, 16)] = (
                            ths[hh].at[k].get(mode="promise_in_bounds"))
                    return 0

                lax.fori_loop(0, _CHUNK // 16, gather_body, 0, unroll=2)

                for hh in range(_HG):
                    h = g * _HG + hh
                    dst = base_out + h * _PLANE + c * _CHUNK
                    pltpu.async_copy(
                        out_v.at[h], out_hbm.at[pl.ds(dst, _CHUNK)], sem_out.at[g])

            nxt = jnp.minimum(c + 2, _NCHUNK - 1)
            pltpu.async_copy(
                dist_hbm.at[pl.ds(base_in + nxt * _CHUNK, _CHUNK)], idx_v, sem_in)
        return 0

    lax.fori_loop(0, _NCHUNK // 2, pair_body, 0)

    for g in range(_NG):
        for hh in range(_HG):
            pltpu.make_async_copy(
                out_v.at[g * _HG + hh],
                out_hbm.at[pl.ds(0, _CHUNK)], sem_out.at[g]).wait()
    for p in range(2):
        pltpu.make_async_copy(
            dist_hbm.at[pl.ds(0, _CHUNK)], idx_bufs[p], sem_ins[p]).wait()


def kernel(dist_matrix, bias_embedding):
    B, N, _ = dist_matrix.shape
    dist_flat = dist_matrix.reshape(B * N * N)
    table_flat = jnp.pad(bias_embedding.T, ((0, 0), (0, 4))).reshape(_H * 16)

    mesh = plsc.VectorSubcoreMesh(core_axis_name="c", subcore_axis_name="s")
    k = functools.partial(
        pl.kernel,
        mesh=mesh,
        out_type=jax.ShapeDtypeStruct((B * _H * N * N,), jnp.float32),
        scratch_types=[
            pltpu.VMEM((_H * 16,), jnp.float32),
            pltpu.VMEM((_CHUNK,), jnp.int32),
            pltpu.VMEM((_CHUNK,), jnp.int32),
            pltpu.VMEM((_H, _CHUNK), jnp.float32),
            pltpu.SemaphoreType.DMA,
            pltpu.SemaphoreType.DMA,
            pltpu.SemaphoreType.DMA((_NG,)),
        ],
        compiler_params=pltpu.CompilerParams(needs_layout_passes=False),
    )(_sc_body)
    out = k(dist_flat, table_flat)
    return out.reshape(B * _H, N, N)


# speedup vs baseline: 5.2009x; 1.4000x over previous
import functools
import jax
import jax.numpy as jnp
from jax import lax
from jax.experimental import pallas as pl
from jax.experimental.pallas import tpu as pltpu
from jax.experimental.pallas import tpu_sc as plsc

_H = 32
_NVALS = 12
_N = 256
_PLANE = _N * _N
_CHUNK = 2048
_NCHUNK = _PLANE // _CHUNK
_HG = 8
_NG = _H // _HG


def _sc_body(dist_hbm, table_hbm, out_hbm, table_v, idx0_v, idx1_v,
             out_v, sem_in0, sem_in1, sem_out):
    nc = 2
    wid = lax.axis_index("s") * nc + lax.axis_index("c")
    pltpu.sync_copy(table_hbm, table_v)

    idx_bufs = (idx0_v, idx1_v)
    sem_ins = (sem_in0, sem_in1)

    pltpu.async_copy(dist_hbm.at[wid, pl.ds(0, _CHUNK)], idx0_v, sem_in0)
    pltpu.async_copy(dist_hbm.at[wid, pl.ds(_CHUNK, _CHUNK)], idx1_v, sem_in1)

    def pair_body(cc, _):
        for p in range(2):
            c = 2 * cc + p
            idx_v = idx_bufs[p]
            sem_in = sem_ins[p]

            pltpu.make_async_copy(
                dist_hbm.at[0, pl.ds(0, _CHUNK)], idx_v, sem_in).wait()

            for g in range(_NG):
                @pl.when(c >= 1)
                def _drain():
                    pltpu.make_async_copy(
                        out_v.at[pl.ds(g * _HG, _HG)],
                        out_hbm.at[pl.ds(0, _HG), pl.ds(0, _CHUNK)],
                        sem_out.at[g]).wait()

                ths = [table_v[pl.ds((g * _HG + hh) * 16, 16)]
                       for hh in range(_HG)]

                def gather_body(i, _):
                    d = idx_v[pl.ds(i * 16, 16)]
                    k = jnp.minimum(jnp.maximum(d + 1, 0), _NVALS - 1)
                    for hh in range(_HG):
                        out_v[g * _HG + hh, pl.ds(i * 16, 16)] = (
                            ths[hh].at[k].get(mode="promise_in_bounds"))
                    return 0

                lax.fori_loop(0, _CHUNK // 16, gather_body, 0, unroll=2)

                pltpu.async_copy(
                    out_v.at[pl.ds(g * _HG, _HG)],
                    out_hbm.at[pl.ds(wid * _H + g * _HG, _HG),
                               pl.ds(c * _CHUNK, _CHUNK)],
                    sem_out.at[g])

            nxt = jnp.minimum(c + 2, _NCHUNK - 1)
            pltpu.async_copy(
                dist_hbm.at[wid, pl.ds(nxt * _CHUNK, _CHUNK)], idx_v, sem_in)
        return 0

    lax.fori_loop(0, _NCHUNK // 2, pair_body, 0)

    for g in range(_NG):
        pltpu.make_async_copy(
            out_v.at[pl.ds(g * _HG, _HG)],
            out_hbm.at[pl.ds(0, _HG), pl.ds(0, _CHUNK)],
            sem_out.at[g]).wait()
    for p in range(2):
        pltpu.make_async_copy(
            dist_hbm.at[0, pl.ds(0, _CHUNK)], idx_bufs[p], sem_ins[p]).wait()


def kernel(dist_matrix, bias_embedding):
    B, N, _ = dist_matrix.shape
    dist_t = dist_matrix.reshape(B, N // 8, 8, 2, 128).transpose(
        0, 1, 3, 2, 4).reshape(B, N * N)
    table_flat = jnp.pad(bias_embedding.T, ((0, 0), (0, 4))).reshape(_H * 16)

    mesh = plsc.VectorSubcoreMesh(core_axis_name="c", subcore_axis_name="s")
    k = functools.partial(
        pl.kernel,
        mesh=mesh,
        out_type=jax.ShapeDtypeStruct((B * _H, N * N), jnp.float32),
        scratch_types=[
            pltpu.VMEM((_H * 16,), jnp.float32),
            pltpu.VMEM((_CHUNK,), jnp.int32),
            pltpu.VMEM((_CHUNK,), jnp.int32),
            pltpu.VMEM((_H, _CHUNK), jnp.float32),
            pltpu.SemaphoreType.DMA,
            pltpu.SemaphoreType.DMA,
            pltpu.SemaphoreType.DMA((_NG,)),
        ],
        compiler_params=pltpu.CompilerParams(needs_layout_passes=False),
    )(_sc_body)
    out = k(dist_t, table_flat)
    return out.reshape(B * _H, N // 8, 2, 8, 128).transpose(
        0, 1, 3, 2, 4).reshape(B * _H, N, N)


# speedup vs baseline: 10.0268x; 1.9279x over previous
import functools
import jax
import jax.numpy as jnp
from jax import lax
from jax.experimental import pallas as pl
from jax.experimental.pallas import tpu as pltpu
from jax.experimental.pallas import tpu_sc as plsc

_H = 32
_NVALS = 12
_N = 256
_R = 8
_NSTRIP = _N // _R
_CHUNK = _R * _N
_HG = 8
_NG = _H // _HG


def _sc_body(dist_hbm, table_hbm, out_hbm, table_v, idx0_v, idx1_v,
             out_v, sem_in0, sem_in1, sem_out):
    nc = 2
    wid = lax.axis_index("s") * nc + lax.axis_index("c")
    pltpu.sync_copy(table_hbm, table_v)

    idx_bufs = (idx0_v, idx1_v)
    sem_ins = (sem_in0, sem_in1)

    pltpu.async_copy(dist_hbm.at[wid, pl.ds(0, _R), :], idx0_v, sem_in0)
    pltpu.async_copy(dist_hbm.at[wid, pl.ds(_R, _R), :], idx1_v, sem_in1)

    def pair_body(cc, _):
        for p in range(2):
            c = 2 * cc + p
            idx_v = idx_bufs[p]
            sem_in = sem_ins[p]

            pltpu.make_async_copy(
                dist_hbm.at[0, pl.ds(0, _R), :], idx_v, sem_in).wait()

            for g in range(_NG):
                @pl.when(c >= 1)
                def _drain():
                    pltpu.make_async_copy(
                        out_v.at[pl.ds(g * _HG, _HG)],
                        out_hbm.at[pl.ds(0, _HG), pl.ds(0, _R), :],
                        sem_out.at[g]).wait()

                ths = [table_v[pl.ds((g * _HG + hh) * 16, 16)]
                       for hh in range(_HG)]

                def gather_body(i, _):
                    r = i >> 4
                    cl = (i & 15) * 16
                    d = idx_v[r, pl.ds(cl, 16)]
                    k = jnp.minimum(jnp.maximum(d + 1, 0), _NVALS - 1)
                    for hh in range(_HG):
                        out_v[g * _HG + hh, r, pl.ds(cl, 16)] = (
                            ths[hh].at[k].get(mode="promise_in_bounds"))
                    return 0

                lax.fori_loop(0, _CHUNK // 16, gather_body, 0, unroll=2)

                pltpu.async_copy(
                    out_v.at[pl.ds(g * _HG, _HG)],
                    out_hbm.at[pl.ds(wid * _H + g * _HG, _HG),
                               pl.ds(c * _R, _R), :],
                    sem_out.at[g])

            nxt = jnp.minimum(c + 2, _NSTRIP - 1)
            pltpu.async_copy(
                dist_hbm.at[wid, pl.ds(nxt * _R, _R), :], idx_v, sem_in)
        return 0

    lax.fori_loop(0, _NSTRIP // 2, pair_body, 0)

    for g in range(_NG):
        pltpu.make_async_copy(
            out_v.at[pl.ds(g * _HG, _HG)],
            out_hbm.at[pl.ds(0, _HG), pl.ds(0, _R), :],
            sem_out.at[g]).wait()
    for p in range(2):
        pltpu.make_async_copy(
            dist_hbm.at[0, pl.ds(0, _R), :], idx_bufs[p], sem_ins[p]).wait()


def kernel(dist_matrix, bias_embedding):
    B, N, _ = dist_matrix.shape
    table_flat = jnp.pad(bias_embedding.T, ((0, 0), (0, 4))).reshape(_H * 16)

    mesh = plsc.VectorSubcoreMesh(core_axis_name="c", subcore_axis_name="s")
    k = functools.partial(
        pl.kernel,
        mesh=mesh,
        out_type=jax.ShapeDtypeStruct((B * _H, N, N), jnp.float32),
        scratch_types=[
            pltpu.VMEM((_H * 16,), jnp.float32),
            pltpu.VMEM((_R, _N), jnp.int32),
            pltpu.VMEM((_R, _N), jnp.int32),
            pltpu.VMEM((_H, _R, _N), jnp.float32),
            pltpu.SemaphoreType.DMA,
            pltpu.SemaphoreType.DMA,
            pltpu.SemaphoreType.DMA((_NG,)),
        ],
        compiler_params=pltpu.CompilerParams(
            needs_layout_passes=False, use_tc_tiling_on_sc=True),
    )(_sc_body)
    return k(dist_matrix, table_flat)
